# baseline (speedup 1.0000x reference)
import functools

import jax
import jax.numpy as jnp
from jax import lax
from jax.experimental import pallas as pl
from jax.experimental.pallas import tpu as pltpu
from jax.experimental.pallas import tpu_sc as plsc

B = 4096
D = 128


def _sc_gather6(tables, idx3):
    info = plsc.get_sparse_core_info()
    nw = info.num_cores * info.num_subcores
    bpw = B // nw
    qrt = bpw // 4
    nbuf = 8
    mesh = plsc.VectorSubcoreMesh(core_axis_name="c", subcore_axis_name="s")
    out_t = tuple(jax.ShapeDtypeStruct((B, D), jnp.float32) for _ in range(6))

    idx_all = idx3.reshape(3, nw, bpw).transpose(1, 0, 2)
    steps = [(t, h) for t in range(6) for h in range(4)]
    ns = len(steps)

    @functools.partial(
        pl.kernel, mesh=mesh, out_type=out_t,
        scratch_types=[
            pltpu.VMEM((3, bpw), jnp.int32),
            pltpu.VMEM((nbuf, qrt, D), jnp.float32),
            pltpu.SemaphoreType.DMA((nbuf,)),
            pltpu.SemaphoreType.DMA((nbuf,)),
        ],
    )
    def k(t0, t1, t2, t3, t4, t5, idx_hbm, o0, o1, o2, o3, o4, o5,
          idx_v, rbuf, gsem, ssem):
        wid = lax.axis_index("s") * info.num_cores + lax.axis_index("c")
        base = wid * bpw
        pltpu.sync_copy(idx_hbm.at[wid], idx_v)
        tabs = [t0, t1, t2, t3, t4, t5]
        outs = [o0, o1, o2, o3, o4, o5]
        g = [None] * ns
        s = [None] * ns

        def launch_scatter(kk):
            t, h = steps[kk]
            g[kk].wait()
            s[kk] = pltpu.async_copy(
                rbuf.at[kk % nbuf],
                outs[t].at[pl.ds(base + h * qrt, qrt)], ssem.at[kk % nbuf])

        for i, (t, h) in enumerate(steps):
            if i >= nbuf:
                s[i - nbuf].wait()
            g[i] = pltpu.async_copy(
                tabs[t].at[idx_v.at[t // 2, pl.ds(h * qrt, qrt)]],
                rbuf.at[i % nbuf], gsem.at[i % nbuf])
            if i >= 1:
                launch_scatter(i - 1)
        launch_scatter(ns - 1)
        for kk in range(ns - nbuf, ns):
            s[kk].wait()

    return k(*tables, idx_all)


def _sc_score(entity_re, entity_im, relation_re, relation_im, idx3):
    info = plsc.get_sparse_core_info()
    nw = info.num_cores * info.num_subcores
    bpw = B // nw
    qrt = bpw // 4
    mesh = plsc.VectorSubcoreMesh(core_axis_name="c", subcore_axis_name="s")

    idx_all = idx3.reshape(3, nw, bpw).transpose(1, 0, 2)

    @functools.partial(
        pl.kernel, mesh=mesh,
        out_type=jax.ShapeDtypeStruct((B, 16), jnp.float32),
        scratch_types=[
            pltpu.VMEM((3, bpw), jnp.int32),
            [pltpu.VMEM((bpw, D), jnp.float32) for _ in range(6)],
            pltpu.VMEM((bpw, 16), jnp.float32),
            pltpu.SemaphoreType.DMA((24,)),
            pltpu.SemaphoreType.DMA,
        ],
    )
    def k(ent_re, ent_im, rel_re, rel_im, idx_hbm, out, idx_v, ops, acc_buf,
          gsem, osem):
        wid = lax.axis_index("s") * info.num_cores + lax.axis_index("c")
        base = wid * bpw
        pltpu.sync_copy(idx_hbm.at[wid], idx_v)
        tabs = [ent_re, ent_im, ent_re, ent_im, rel_re, rel_im]
        descs = []
        for q in range(4):
            for i in range(6):
                descs.append(pltpu.async_copy(
                    tabs[i].at[idx_v.at[i // 2, pl.ds(q * qrt, qrt)]],
                    ops[i].at[pl.ds(q * qrt, qrt)],
                    gsem.at[q * 6 + i]))

        def row_body(r, u, lo):
            row = lo + r
            acc = jnp.zeros((16,), jnp.float32)
            for c in range(8):
                sl = pl.ds(c * 16, 16)
                hre = ops[0][row, sl]
                him = ops[1][row, sl]
                tre = ops[2][row, sl]
                tim = ops[3][row, sl]
                rre = ops[4][row, sl]
                rim = ops[5][row, sl]
                acc = (acc + hre * (rre * tre + rim * tim)
                       + him * (rre * tim - rim * tre))
            acc_buf[row, :] = acc
            return u

        for q in range(4):
            for i in range(6):
                descs[q * 6 + i].wait()
            lax.fori_loop(0, qrt,
                          lambda r, u, lo=q * qrt: row_body(r, u, lo), 0)
        pltpu.async_copy(acc_buf, out.at[pl.ds(base, bpw)], osem).wait()

    return k(entity_re, entity_im, relation_re, relation_im, idx_all)


def _tc_gram(ure, uim, ire, iim, bre, bim, reg_ref):
    def gram_norm(a_ref, b_ref):
        a = a_ref[...]
        b = b_ref[...]
        dn = (((0,), (0,)), ((), ()))
        g = (lax.dot_general(a, a, dn, preferred_element_type=jnp.float32)
             + lax.dot_general(b, b, dn, preferred_element_type=jnp.float32))
        return jnp.sqrt(jnp.sum(g * g))

    reg = gram_norm(ure, uim) + gram_norm(ire, iim) + gram_norm(bre, bim)
    reg_ref[...] = reg.reshape(1, 1)


def _tc_finish(acc, score_ref):
    score_ref[...] = jax.nn.sigmoid(jnp.sum(acc[...], axis=1))


def kernel(entity_re, entity_im, relation_re, relation_im,
           head, tail, relation, reg_user, reg_item, reg_brand):
    idx_reg = jnp.stack([reg_user, reg_item, reg_brand])
    idx_score = jnp.stack([head, tail, relation])
    ure, uim, ire, iim, bre, bim = _sc_gather6(
        [entity_re, entity_im, entity_re, entity_im, entity_re, entity_im],
        idx_reg)
    acc = _sc_score(entity_re, entity_im, relation_re, relation_im,
                    idx_score)
    reg = pl.pallas_call(
        _tc_gram,
        out_shape=jax.ShapeDtypeStruct((1, 1), jnp.float32),
    )(ure, uim, ire, iim, bre, bim)
    score = pl.pallas_call(
        _tc_finish,
        out_shape=jax.ShapeDtypeStruct((B,), jnp.float32),
    )(acc)
    return score, reg[0, 0]


# speedup vs baseline: 1.0757x; 1.0757x over previous
import functools

import jax
import jax.numpy as jnp
from jax import lax
from jax.experimental import pallas as pl
from jax.experimental.pallas import tpu as pltpu
from jax.experimental.pallas import tpu_sc as plsc

B = 4096
D = 128


def _sc_gather6(tables, idx3):
    info = plsc.get_sparse_core_info()
    nw = info.num_cores * info.num_subcores
    bpw = B // nw
    nbuf = 6
    nt = 6
    mesh = plsc.VectorSubcoreMesh(core_axis_name="c", subcore_axis_name="s")
    out_t = tuple(jax.ShapeDtypeStruct((B, D), jnp.float32) for _ in range(nt))

    idx_all = idx3.reshape(3, nw, bpw).transpose(1, 0, 2)

    @functools.partial(
        pl.kernel, mesh=mesh, out_type=out_t,
        scratch_types=[
            pltpu.VMEM((3, bpw), jnp.int32),
            pltpu.VMEM((nbuf, bpw, D), jnp.float32),
            pltpu.SemaphoreType.DMA((nbuf,)),
            pltpu.SemaphoreType.DMA((nbuf,)),
        ],
    )
    def k(t0, t1, t2, t3, t4, t5, idx_hbm, o0, o1, o2, o3, o4, o5,
          idx_v, rows, gsem, ssem):
        wid = lax.axis_index("s") * info.num_cores + lax.axis_index("c")
        base = wid * bpw
        pltpu.sync_copy(idx_hbm.at[wid], idx_v)
        tabs = [t0, t1, t2, t3, t4, t5]
        outs = [o0, o1, o2, o3, o4, o5]
        g = [None] * nt
        s = [None] * nt

        def launch_scatter(kk):
            b = kk % nbuf
            g[kk].wait()
            s[kk] = pltpu.async_copy(
                rows.at[b], outs[kk].at[pl.ds(base, bpw)], ssem.at[b])

        for t in range(nt):
            b = t % nbuf
            if t >= nbuf:
                s[t - nbuf].wait()
            g[t] = pltpu.async_copy(tabs[t].at[idx_v.at[t // 2]], rows.at[b],
                                    gsem.at[b])
            if t >= 1:
                launch_scatter(t - 1)
        launch_scatter(nt - 1)
        for kk in range(max(nt - nbuf, 0), nt):
            s[kk].wait()

    return k(*tables, idx_all)


def _sc_score(entity_re, entity_im, relation_re, relation_im, idx3):
    info = plsc.get_sparse_core_info()
    nw = info.num_cores * info.num_subcores
    bpw = B // nw
    qrt = bpw // 4
    mesh = plsc.VectorSubcoreMesh(core_axis_name="c", subcore_axis_name="s")

    idx_all = idx3.reshape(3, nw, bpw).transpose(1, 0, 2)

    @functools.partial(
        pl.kernel, mesh=mesh,
        out_type=jax.ShapeDtypeStruct((B, 16), jnp.float32),
        scratch_types=[
            pltpu.VMEM((3, bpw), jnp.int32),
            [pltpu.VMEM((bpw, D), jnp.float32) for _ in range(6)],
            pltpu.VMEM((bpw, 16), jnp.float32),
            pltpu.SemaphoreType.DMA((24,)),
            pltpu.SemaphoreType.DMA,
        ],
    )
    def k(ent_re, ent_im, rel_re, rel_im, idx_hbm, out, idx_v, ops, acc_buf,
          gsem, osem):
        wid = lax.axis_index("s") * info.num_cores + lax.axis_index("c")
        base = wid * bpw
        pltpu.sync_copy(idx_hbm.at[wid], idx_v)
        tabs = [ent_re, ent_im, ent_re, ent_im, rel_re, rel_im]
        descs = []
        for q in range(4):
            for i in range(6):
                descs.append(pltpu.async_copy(
                    tabs[i].at[idx_v.at[i // 2, pl.ds(q * qrt, qrt)]],
                    ops[i].at[pl.ds(q * qrt, qrt)],
                    gsem.at[q * 6 + i]))

        def row_body(r, u, lo):
            row = lo + r
            acc = jnp.zeros((16,), jnp.float32)
            for c in range(8):
                sl = pl.ds(c * 16, 16)
                hre = ops[0][row, sl]
                him = ops[1][row, sl]
                tre = ops[2][row, sl]
                tim = ops[3][row, sl]
                rre = ops[4][row, sl]
                rim = ops[5][row, sl]
                acc = (acc + hre * (rre * tre + rim * tim)
                       + him * (rre * tim - rim * tre))
            acc_buf[row, :] = acc
            return u

        for q in range(4):
            for i in range(6):
                descs[q * 6 + i].wait()
            lax.fori_loop(0, qrt,
                          lambda r, u, lo=q * qrt: row_body(r, u, lo), 0)
        pltpu.async_copy(acc_buf, out.at[pl.ds(base, bpw)], osem).wait()

    return k(entity_re, entity_im, relation_re, relation_im, idx_all)


def _tc_gram(ure, uim, ire, iim, bre, bim, reg_ref):
    def gram_norm(a_ref, b_ref):
        a = a_ref[...]
        b = b_ref[...]
        dn = (((0,), (0,)), ((), ()))
        g = (lax.dot_general(a, a, dn, preferred_element_type=jnp.float32)
             + lax.dot_general(b, b, dn, preferred_element_type=jnp.float32))
        return jnp.sqrt(jnp.sum(g * g))

    reg = gram_norm(ure, uim) + gram_norm(ire, iim) + gram_norm(bre, bim)
    reg_ref[...] = reg.reshape(1, 1)


def _tc_finish(acc, score_ref):
    score_ref[...] = jax.nn.sigmoid(jnp.sum(acc[...], axis=1))


def kernel(entity_re, entity_im, relation_re, relation_im,
           head, tail, relation, reg_user, reg_item, reg_brand):
    idx_reg = jnp.stack([reg_user, reg_item, reg_brand])
    idx_score = jnp.stack([head, tail, relation])
    ure, uim, ire, iim, bre, bim = _sc_gather6(
        [entity_re, entity_im, entity_re, entity_im, entity_re, entity_im],
        idx_reg)
    acc = _sc_score(entity_re, entity_im, relation_re, relation_im,
                    idx_score)
    reg = pl.pallas_call(
        _tc_gram,
        out_shape=jax.ShapeDtypeStruct((1, 1), jnp.float32),
    )(ure, uim, ire, iim, bre, bim)
    score = pl.pallas_call(
        _tc_finish,
        out_shape=jax.ShapeDtypeStruct((B,), jnp.float32),
    )(acc)
    return score, reg[0, 0]


# speedup vs baseline: 1.1090x; 1.0309x over previous
import functools

import jax
import jax.numpy as jnp
from jax import lax
from jax.experimental import pallas as pl
from jax.experimental.pallas import tpu as pltpu
from jax.experimental.pallas import tpu_sc as plsc

B = 4096
D = 128


def _sc_gather6(tables, idx3):
    info = plsc.get_sparse_core_info()
    nw = info.num_cores * info.num_subcores
    bpw = B // nw
    nbuf = 6
    nt = 6
    mesh = plsc.VectorSubcoreMesh(core_axis_name="c", subcore_axis_name="s")
    out_t = tuple(jax.ShapeDtypeStruct((B, D), jnp.float32) for _ in range(nt))

    idx_all = idx3.reshape(3, nw, bpw).transpose(1, 0, 2)

    @functools.partial(
        pl.kernel, mesh=mesh, out_type=out_t,
        scratch_types=[
            pltpu.VMEM((3, bpw), jnp.int32),
            pltpu.VMEM((nbuf, bpw, D), jnp.float32),
            pltpu.SemaphoreType.DMA((nbuf,)),
            pltpu.SemaphoreType.DMA((nbuf,)),
        ],
    )
    def k(t0, t1, t2, t3, t4, t5, idx_hbm, o0, o1, o2, o3, o4, o5,
          idx_v, rows, gsem, ssem):
        wid = lax.axis_index("s") * info.num_cores + lax.axis_index("c")
        base = wid * bpw
        pltpu.sync_copy(idx_hbm.at[wid], idx_v)
        tabs = [t0, t1, t2, t3, t4, t5]
        outs = [o0, o1, o2, o3, o4, o5]
        g = [None] * nt
        s = [None] * nt

        def launch_scatter(kk):
            b = kk % nbuf
            g[kk].wait()
            s[kk] = pltpu.async_copy(
                rows.at[b], outs[kk].at[pl.ds(base, bpw)], ssem.at[b])

        for t in range(nt):
            b = t % nbuf
            if t >= nbuf:
                s[t - nbuf].wait()
            g[t] = pltpu.async_copy(tabs[t].at[idx_v.at[t // 2]], rows.at[b],
                                    gsem.at[b])
            if t >= nbuf - 1:
                launch_scatter(t - (nbuf - 1))
        for kk in range(max(nt - (nbuf - 1), 0), nt):
            launch_scatter(kk)
        for kk in range(max(nt - nbuf, 0), nt):
            s[kk].wait()

    return k(*tables, idx_all)


def _sc_score(entity_re, entity_im, relation_re, relation_im, idx3):
    info = plsc.get_sparse_core_info()
    nw = info.num_cores * info.num_subcores
    bpw = B // nw
    qrt = bpw // 4
    mesh = plsc.VectorSubcoreMesh(core_axis_name="c", subcore_axis_name="s")

    idx_all = idx3.reshape(3, nw, bpw).transpose(1, 0, 2)

    @functools.partial(
        pl.kernel, mesh=mesh,
        out_type=jax.ShapeDtypeStruct((B, 16), jnp.float32),
        scratch_types=[
            pltpu.VMEM((3, bpw), jnp.int32),
            [pltpu.VMEM((bpw, D), jnp.float32) for _ in range(6)],
            pltpu.VMEM((bpw, 16), jnp.float32),
            pltpu.SemaphoreType.DMA((24,)),
            pltpu.SemaphoreType.DMA,
        ],
    )
    def k(ent_re, ent_im, rel_re, rel_im, idx_hbm, out, idx_v, ops, acc_buf,
          gsem, osem):
        wid = lax.axis_index("s") * info.num_cores + lax.axis_index("c")
        base = wid * bpw
        pltpu.sync_copy(idx_hbm.at[wid], idx_v)
        tabs = [ent_re, ent_im, ent_re, ent_im, rel_re, rel_im]
        descs = []
        for q in range(4):
            for i in range(6):
                descs.append(pltpu.async_copy(
                    tabs[i].at[idx_v.at[i // 2, pl.ds(q * qrt, qrt)]],
                    ops[i].at[pl.ds(q * qrt, qrt)],
                    gsem.at[q * 6 + i]))

        def row_body(r, u, lo):
            row = lo + r
            acc = jnp.zeros((16,), jnp.float32)
            for c in range(8):
                sl = pl.ds(c * 16, 16)
                hre = ops[0][row, sl]
                him = ops[1][row, sl]
                tre = ops[2][row, sl]
                tim = ops[3][row, sl]
                rre = ops[4][row, sl]
                rim = ops[5][row, sl]
                acc = (acc + hre * (rre * tre + rim * tim)
                       + him * (rre * tim - rim * tre))
            acc_buf[row, :] = acc
            return u

        for q in range(4):
            for i in range(6):
                descs[q * 6 + i].wait()
            lax.fori_loop(0, qrt,
                          lambda r, u, lo=q * qrt: row_body(r, u, lo), 0)
        pltpu.async_copy(acc_buf, out.at[pl.ds(base, bpw)], osem).wait()

    return k(entity_re, entity_im, relation_re, relation_im, idx_all)


def _tc_gram(ure, uim, ire, iim, bre, bim, reg_ref):
    def gram_norm(a_ref, b_ref):
        a = a_ref[...]
        b = b_ref[...]
        dn = (((0,), (0,)), ((), ()))
        g = (lax.dot_general(a, a, dn, preferred_element_type=jnp.float32)
             + lax.dot_general(b, b, dn, preferred_element_type=jnp.float32))
        return jnp.sqrt(jnp.sum(g * g))

    reg = gram_norm(ure, uim) + gram_norm(ire, iim) + gram_norm(bre, bim)
    reg_ref[...] = reg.reshape(1, 1)


def _tc_finish(acc, score_ref):
    score_ref[...] = jax.nn.sigmoid(jnp.sum(acc[...], axis=1))


def kernel(entity_re, entity_im, relation_re, relation_im,
           head, tail, relation, reg_user, reg_item, reg_brand):
    idx_reg = jnp.stack([reg_user, reg_item, reg_brand])
    idx_score = jnp.stack([head, tail, relation])
    ure, uim, ire, iim, bre, bim = _sc_gather6(
        [entity_re, entity_im, entity_re, entity_im, entity_re, entity_im],
        idx_reg)
    acc = _sc_score(entity_re, entity_im, relation_re, relation_im,
                    idx_score)
    reg = pl.pallas_call(
        _tc_gram,
        out_shape=jax.ShapeDtypeStruct((1, 1), jnp.float32),
    )(ure, uim, ire, iim, bre, bim)
    score = pl.pallas_call(
        _tc_finish,
        out_shape=jax.ShapeDtypeStruct((B,), jnp.float32),
    )(acc)
    return score, reg[0, 0]
